# Initial kernel scaffold; baseline (speedup 1.0000x reference)
import jax
import jax.numpy as jnp
from jax.experimental import pallas as pl


def kernel(sigmas, y, py):
    raise NotImplementedError("write your pallas kernel here")


# speedup vs baseline: 168.8335x; 168.8335x over previous
import jax
import jax.numpy as jnp
from jax.experimental import pallas as pl
from jax.experimental.pallas import tpu as pltpu

_N = 4096
_BLOCK = 256
_NB = _N // _BLOCK
_NUM_BINS = 10


def _sig2(u):
    a = u / 200.0
    sig = jnp.sqrt(-2.0 / jnp.log(1.0 - a * a))
    return sig * sig


def _second_extreme(flat, m, is_max):
    eq = flat == m
    cnt = jnp.sum(eq.astype(jnp.float32))
    fill = -jnp.inf if is_max else jnp.inf
    red = jnp.max if is_max else jnp.min
    other = red(jnp.where(eq, jnp.float32(fill), flat))
    return jnp.where(cnt >= 2.0, m, other)


def _pair_kernel(sig_row_ref, sig_all_ref, y_ref, yt_ref, py_ref, pyt_ref,
                 out_ref, acc_ref):
    b = pl.program_id(0)
    i0 = b * _BLOCK

    sig_all = sig_all_ref[:, :]
    sig_row = sig_row_ref[pl.ds(i0, _BLOCK), :]

    m1 = jnp.max(sig_all)
    m2 = _second_extreme(sig_all, m1, True)
    n1 = jnp.min(sig_all)
    n2 = _second_extreme(sig_all, n1, False)
    begin = _sig2(m1 + m2)
    end = _sig2(n1 + n2)
    interval = (end - begin) / _NUM_BINS
    left9 = begin + 9 * interval
    left10 = begin + 10 * interval + 1.0

    d2_gt = jnp.zeros((_BLOCK, _N), jnp.float32)
    d2_pr = jnp.zeros((_BLOCK, _N), jnp.float32)
    for c in range(3):
        yr = y_ref[pl.ds(i0, _BLOCK), c:c + 1]
        ya = yt_ref[c:c + 1, :]
        dy = yr - ya
        d2_gt = d2_gt + dy * dy
        pr = py_ref[pl.ds(i0, _BLOCK), c:c + 1]
        pa = pyt_ref[c:c + 1, :]
        dp = pr - pa
        d2_pr = d2_pr + dp * dp
    diff = jnp.sqrt(d2_gt) - jnp.sqrt(d2_pr)
    t = diff * diff

    s = _sig2(sig_row + sig_all)

    cols = jax.lax.broadcasted_iota(jnp.int32, (_BLOCK, _N), 1)
    rows = i0 + jax.lax.broadcasted_iota(jnp.int32, (_BLOCK, _N), 0)
    mask = (s >= left9) & (s <= left10) & (cols != rows)
    maskf = mask.astype(jnp.float32)

    p_s = jnp.sum(s * maskf)
    p_t = jnp.sum(t * maskf)
    p_c = jnp.sum(maskf)

    @pl.when(b == 0)
    def _init():
        acc_ref[0] = 0.0
        acc_ref[1] = 0.0
        acc_ref[2] = 0.0

    acc_ref[0] += p_s
    acc_ref[1] += p_t
    acc_ref[2] += p_c

    @pl.when(b == _NB - 1)
    def _fini():
        cnt = acc_ref[2]
        mvar = jnp.sqrt(acc_ref[0] / cnt)
        rmse = jnp.sqrt(acc_ref[1] / cnt)
        val = jnp.abs(mvar - rmse) / mvar
        out_ref[:, :] = jnp.full((1, 1), val, jnp.float32)


def kernel(sigmas, y, py):
    sig_row = sigmas.reshape(_N, 1)
    sig_all = sigmas.reshape(1, _N)
    yt = y.T
    pyt = py.T
    full = lambda shape: pl.BlockSpec(shape, lambda b: (0,) * len(shape))
    out = pl.pallas_call(
        _pair_kernel,
        grid=(_NB,),
        in_specs=[
            full((_N, 1)),
            full((1, _N)),
            full((_N, 3)),
            full((3, _N)),
            full((_N, 3)),
            full((3, _N)),
        ],
        out_specs=full((1, 1)),
        out_shape=jax.ShapeDtypeStruct((1, 1), jnp.float32),
        scratch_shapes=[pltpu.SMEM((3,), jnp.float32)],
        compiler_params=pltpu.CompilerParams(
            dimension_semantics=("arbitrary",)),
    )(sig_row, sig_all, y, yt, py, pyt)
    return out[0, 0]


# speedup vs baseline: 190.8598x; 1.1305x over previous
import jax
import jax.numpy as jnp
from jax.experimental import pallas as pl
from jax.experimental.pallas import tpu as pltpu

_N = 4096
_BLOCK = 256
_NB = _N // _BLOCK
_NUM_BINS = 10


def _sig2(u):
    a = u / 200.0
    sig = jnp.sqrt(-2.0 / jnp.log(1.0 - a * a))
    return sig * sig


def _second_extreme(flat, m, is_max):
    eq = flat == m
    cnt = jnp.sum(eq.astype(jnp.float32))
    fill = -jnp.inf if is_max else jnp.inf
    red = jnp.max if is_max else jnp.min
    other = red(jnp.where(eq, jnp.float32(fill), flat))
    return jnp.where(cnt >= 2.0, m, other)


def _pair_kernel(sig_row_ref, sig_all_ref, y_ref, yt_ref, py_ref, pyt_ref,
                 out_ref, acc_ref):
    b = pl.program_id(0)
    i0 = b * _BLOCK

    sig_all = sig_all_ref[:, :]
    sig_row = sig_row_ref[pl.ds(i0, _BLOCK), :]

    m1 = jnp.max(sig_all)
    m2 = _second_extreme(sig_all, m1, True)
    n1 = jnp.min(sig_all)
    n2 = _second_extreme(sig_all, n1, False)
    begin = _sig2(m1 + m2)
    end = _sig2(n1 + n2)
    interval = (end - begin) / _NUM_BINS
    left9 = begin + 9 * interval
    left10 = begin + 10 * interval + 1.0

    hp = jax.lax.Precision.HIGHEST
    dn = (((1,), (0,)), ((), ()))
    yb = y_ref[pl.ds(i0, _BLOCK), :]
    ya = yt_ref[:, :]
    g_gt = jax.lax.dot_general(yb, ya, dn, precision=hp)
    rr_gt = jnp.sum(yb * yb, axis=1, keepdims=True)
    rc_gt = jnp.sum(ya * ya, axis=0, keepdims=True)
    d2_gt = jnp.maximum(rr_gt + rc_gt - 2.0 * g_gt, 0.0)
    pb = py_ref[pl.ds(i0, _BLOCK), :]
    pa = pyt_ref[:, :]
    g_pr = jax.lax.dot_general(pb, pa, dn, precision=hp)
    rr_pr = jnp.sum(pb * pb, axis=1, keepdims=True)
    rc_pr = jnp.sum(pa * pa, axis=0, keepdims=True)
    d2_pr = jnp.maximum(rr_pr + rc_pr - 2.0 * g_pr, 0.0)
    t = d2_gt + d2_pr - 2.0 * jnp.sqrt(d2_gt * d2_pr)

    s = _sig2(sig_row + sig_all)

    cols = jax.lax.broadcasted_iota(jnp.int32, (_BLOCK, _N), 1)
    rows = i0 + jax.lax.broadcasted_iota(jnp.int32, (_BLOCK, _N), 0)
    mask = (s >= left9) & (s <= left10) & (cols != rows)
    maskf = mask.astype(jnp.float32)

    p_s = jnp.sum(s * maskf)
    p_t = jnp.sum(t * maskf)
    p_c = jnp.sum(maskf)

    @pl.when(b == 0)
    def _init():
        acc_ref[0] = 0.0
        acc_ref[1] = 0.0
        acc_ref[2] = 0.0

    acc_ref[0] += p_s
    acc_ref[1] += p_t
    acc_ref[2] += p_c

    @pl.when(b == _NB - 1)
    def _fini():
        cnt = acc_ref[2]
        mvar = jnp.sqrt(acc_ref[0] / cnt)
        rmse = jnp.sqrt(acc_ref[1] / cnt)
        val = jnp.abs(mvar - rmse) / mvar
        out_ref[:, :] = jnp.full((1, 1), val, jnp.float32)


def kernel(sigmas, y, py):
    sig_row = sigmas.reshape(_N, 1)
    sig_all = sigmas.reshape(1, _N)
    y8 = jnp.pad(y, ((0, 0), (0, 5)))
    py8 = jnp.pad(py, ((0, 0), (0, 5)))
    yt = y8.T
    pyt = py8.T
    full = lambda shape: pl.BlockSpec(shape, lambda b: (0,) * len(shape))
    out = pl.pallas_call(
        _pair_kernel,
        grid=(_NB,),
        in_specs=[
            full((_N, 1)),
            full((1, _N)),
            full((_N, 8)),
            full((8, _N)),
            full((_N, 8)),
            full((8, _N)),
        ],
        out_specs=full((1, 1)),
        out_shape=jax.ShapeDtypeStruct((1, 1), jnp.float32),
        scratch_shapes=[pltpu.SMEM((3,), jnp.float32)],
        compiler_params=pltpu.CompilerParams(
            dimension_semantics=("arbitrary",)),
    )(sig_row, sig_all, y8, yt, py8, pyt)
    return out[0, 0]


# speedup vs baseline: 249.3529x; 1.3065x over previous
import jax
import jax.numpy as jnp
from jax.experimental import pallas as pl
from jax.experimental.pallas import tpu as pltpu

_N = 4096
_B = 512
_NB = _N // _B
_NUM_BINS = 10


def _sig2(u):
    a = u / 200.0
    sig = jnp.sqrt(-2.0 / jnp.log(1.0 - a * a))
    return sig * sig


def _second_extreme(flat, m, is_max):
    eq = flat == m
    cnt = jnp.sum(eq.astype(jnp.float32))
    fill = -jnp.inf if is_max else jnp.inf
    red = jnp.max if is_max else jnp.min
    other = red(jnp.where(eq, jnp.float32(fill), flat))
    return jnp.where(cnt >= 2.0, m, other)


def _pair_kernel(sig_row_ref, sig_all_ref, y_ref, yt_ref, py_ref, pyt_ref,
                 out_ref, acc_ref, thr_ref, rc_ref):
    bi = pl.program_id(0)
    cbi = pl.program_id(1)

    @pl.when((bi == 0) & (cbi == 0))
    def _prologue():
        sig_all = sig_all_ref[:, :]
        m1 = jnp.max(sig_all)
        m2 = _second_extreme(sig_all, m1, True)
        n1 = jnp.min(sig_all)
        n2 = _second_extreme(sig_all, n1, False)
        begin = _sig2(m1 + m2)
        end = _sig2(n1 + n2)
        interval = (end - begin) / _NUM_BINS
        thr_ref[0] = begin + 9 * interval
        thr_ref[1] = begin + 10 * interval + 1.0
        ya = yt_ref[:, :]
        pa = pyt_ref[:, :]
        rc_ref[0:1, :] = jnp.sum(ya * ya, axis=0, keepdims=True)
        rc_ref[1:2, :] = jnp.sum(pa * pa, axis=0, keepdims=True)
        acc_ref[0] = 0.0
        acc_ref[1] = 0.0
        acc_ref[2] = 0.0

    @pl.when(cbi >= bi)
    def _body():
        left9 = thr_ref[0]
        left10 = thr_ref[1]
        i0 = bi * _B
        j0 = cbi * _B
        hp = jax.lax.Precision.HIGHEST
        dn = (((1,), (0,)), ((), ()))

        yb = y_ref[pl.ds(i0, _B), :]
        ya = yt_ref[:, pl.ds(j0, _B)]
        g_gt = jax.lax.dot_general(yb, ya, dn, precision=hp)
        rr_gt = jnp.sum(yb * yb, axis=1, keepdims=True)
        d2_gt = jnp.maximum(rr_gt + rc_ref[0:1, pl.ds(j0, _B)] - 2.0 * g_gt,
                            0.0)
        pb = py_ref[pl.ds(i0, _B), :]
        pa = pyt_ref[:, pl.ds(j0, _B)]
        g_pr = jax.lax.dot_general(pb, pa, dn, precision=hp)
        rr_pr = jnp.sum(pb * pb, axis=1, keepdims=True)
        d2_pr = jnp.maximum(rr_pr + rc_ref[1:2, pl.ds(j0, _B)] - 2.0 * g_pr,
                            0.0)
        t = d2_gt + d2_pr - 2.0 * jnp.sqrt(d2_gt * d2_pr)

        s = _sig2(sig_row_ref[pl.ds(i0, _B), :] +
                  sig_all_ref[:, pl.ds(j0, _B)])
        in_bin = (s >= left9) & (s <= left10)

        @pl.when(cbi > bi)
        def _offdiag():
            acc_ref[0] += 2.0 * jnp.sum(jnp.where(in_bin, s, 0.0))
            acc_ref[1] += 2.0 * jnp.sum(jnp.where(in_bin, t, 0.0))
            acc_ref[2] += 2.0 * jnp.sum(jnp.where(in_bin, 1.0, 0.0))

        @pl.when(cbi == bi)
        def _diag():
            cols = j0 + jax.lax.broadcasted_iota(jnp.int32, (_B, _B), 1)
            rows = i0 + jax.lax.broadcasted_iota(jnp.int32, (_B, _B), 0)
            m = in_bin & (cols != rows)
            acc_ref[0] += jnp.sum(jnp.where(m, s, 0.0))
            acc_ref[1] += jnp.sum(jnp.where(m, t, 0.0))
            acc_ref[2] += jnp.sum(jnp.where(m, 1.0, 0.0))

    @pl.when((bi == _NB - 1) & (cbi == _NB - 1))
    def _fini():
        cnt = acc_ref[2]
        mvar = jnp.sqrt(acc_ref[0] / cnt)
        rmse = jnp.sqrt(acc_ref[1] / cnt)
        val = jnp.abs(mvar - rmse) / mvar
        out_ref[:, :] = jnp.full((1, 1), val, jnp.float32)


def kernel(sigmas, y, py):
    sig_row = sigmas.reshape(_N, 1)
    sig_all = sigmas.reshape(1, _N)
    y8 = jnp.pad(y, ((0, 0), (0, 5)))
    py8 = jnp.pad(py, ((0, 0), (0, 5)))
    yt = y8.T
    pyt = py8.T
    full = lambda shape: pl.BlockSpec(shape, lambda b, c: (0,) * len(shape))
    out = pl.pallas_call(
        _pair_kernel,
        grid=(_NB, _NB),
        in_specs=[
            full((_N, 1)),
            full((1, _N)),
            full((_N, 8)),
            full((8, _N)),
            full((_N, 8)),
            full((8, _N)),
        ],
        out_specs=full((1, 1)),
        out_shape=jax.ShapeDtypeStruct((1, 1), jnp.float32),
        scratch_shapes=[
            pltpu.SMEM((3,), jnp.float32),
            pltpu.SMEM((2,), jnp.float32),
            pltpu.VMEM((2, _N), jnp.float32),
        ],
        compiler_params=pltpu.CompilerParams(
            dimension_semantics=("arbitrary", "arbitrary")),
    )(sig_row, sig_all, y8, yt, py8, pyt)
    return out[0, 0]


# speedup vs baseline: 273.5416x; 1.0970x over previous
import jax
import jax.numpy as jnp
import numpy as np
from jax.experimental import pallas as pl
from jax.experimental.pallas import tpu as pltpu

_N = 4096
_B = 512
_NB = _N // _B
_NSTEPS = _NB * (_NB + 1) // 2
_NUM_BINS = 10


def _sig2(u):
    a = u / 200.0
    sig = jnp.sqrt(-2.0 / jnp.log(1.0 - a * a))
    return sig * sig


def _second_extreme(flat, m, is_max):
    eq = flat == m
    cnt = jnp.sum(eq.astype(jnp.float32))
    fill = -jnp.inf if is_max else jnp.inf
    red = jnp.max if is_max else jnp.min
    other = red(jnp.where(eq, jnp.float32(fill), flat))
    return jnp.where(cnt >= 2.0, m, other)


def _pair_kernel(bs_ref, cs_ref, sig_row_ref, sig_all_ref, y_ref, yt_ref,
                 py_ref, pyt_ref, out_ref, thr_ref, rc_ref, rr_ref, vacc_ref):
    step = pl.program_id(0)
    bi = bs_ref[step]
    cbi = cs_ref[step]

    @pl.when(step == 0)
    def _prologue():
        sig_all = sig_all_ref[:, :]
        m1 = jnp.max(sig_all)
        m2 = _second_extreme(sig_all, m1, True)
        n1 = jnp.min(sig_all)
        n2 = _second_extreme(sig_all, n1, False)
        begin = _sig2(m1 + m2)
        end = _sig2(n1 + n2)
        interval = (end - begin) / _NUM_BINS
        thr_ref[0] = begin + 9 * interval
        thr_ref[1] = begin + 10 * interval + 1.0
        ya = yt_ref[:, :]
        pa = pyt_ref[:, :]
        rc_ref[0:1, :] = jnp.sum(ya * ya, axis=0, keepdims=True)
        rc_ref[1:2, :] = jnp.sum(pa * pa, axis=0, keepdims=True)
        yb = y_ref[:, :]
        pb = py_ref[:, :]
        rr_ref[:, 0:1] = jnp.sum(yb * yb, axis=1, keepdims=True)
        rr_ref[:, 1:2] = jnp.sum(pb * pb, axis=1, keepdims=True)
        vacc_ref[:, :] = jnp.zeros((24, _B), jnp.float32)

    left9 = thr_ref[0]
    left10 = thr_ref[1]
    i0 = bi * _B
    j0 = cbi * _B
    hp = jax.lax.Precision.HIGHEST
    dn = (((1,), (0,)), ((), ()))

    yb = y_ref[pl.ds(i0, _B), :]
    ya = yt_ref[:, pl.ds(j0, _B)]
    g_gt = jax.lax.dot_general(yb, ya, dn, precision=hp)
    d2_gt = jnp.maximum(
        rr_ref[pl.ds(i0, _B), 0:1] + rc_ref[0:1, pl.ds(j0, _B)] - 2.0 * g_gt,
        0.0)
    pb = py_ref[pl.ds(i0, _B), :]
    pa = pyt_ref[:, pl.ds(j0, _B)]
    g_pr = jax.lax.dot_general(pb, pa, dn, precision=hp)
    d2_pr = jnp.maximum(
        rr_ref[pl.ds(i0, _B), 1:2] + rc_ref[1:2, pl.ds(j0, _B)] - 2.0 * g_pr,
        0.0)
    t = d2_gt + d2_pr - 2.0 * jnp.sqrt(d2_gt * d2_pr)

    s = _sig2(sig_row_ref[pl.ds(i0, _B), :] +
              sig_all_ref[:, pl.ds(j0, _B)])

    offd = cbi != bi
    cols = j0 + jax.lax.broadcasted_iota(jnp.int32, (_B, _B), 1)
    rows = i0 + jax.lax.broadcasted_iota(jnp.int32, (_B, _B), 0)
    m = (s >= left9) & (s <= left10) & ((cols != rows) | offd)
    w = jnp.where(offd, 2.0, 1.0)

    def _fold(x):
        return jnp.sum(x.reshape(_B // 8, 8, _B), axis=0)

    vacc_ref[0:8, :] += w * _fold(jnp.where(m, s, 0.0))
    vacc_ref[8:16, :] += w * _fold(jnp.where(m, t, 0.0))
    vacc_ref[16:24, :] += w * _fold(jnp.where(m, 1.0, 0.0))

    @pl.when(step == _NSTEPS - 1)
    def _fini():
        cnt = jnp.sum(vacc_ref[16:24, :])
        mvar = jnp.sqrt(jnp.sum(vacc_ref[0:8, :]) / cnt)
        rmse = jnp.sqrt(jnp.sum(vacc_ref[8:16, :]) / cnt)
        val = jnp.abs(mvar - rmse) / mvar
        out_ref[:, :] = jnp.full((1, 1), val, jnp.float32)


_BS = np.array([b for b in range(_NB) for c in range(b, _NB)], np.int32)
_CS = np.array([c for b in range(_NB) for c in range(b, _NB)], np.int32)


def kernel(sigmas, y, py):
    sig_row = sigmas.reshape(_N, 1)
    sig_all = sigmas.reshape(1, _N)
    y8 = jnp.pad(y, ((0, 0), (0, 5)))
    py8 = jnp.pad(py, ((0, 0), (0, 5)))
    yt = y8.T
    pyt = py8.T
    full = lambda shape: pl.BlockSpec(shape, lambda *_: (0,) * len(shape))
    out = pl.pallas_call(
        _pair_kernel,
        grid_spec=pltpu.PrefetchScalarGridSpec(
            num_scalar_prefetch=2,
            grid=(_NSTEPS,),
            in_specs=[
                full((_N, 1)),
                full((1, _N)),
                full((_N, 8)),
                full((8, _N)),
                full((_N, 8)),
                full((8, _N)),
            ],
            out_specs=full((1, 1)),
            scratch_shapes=[
                pltpu.SMEM((2,), jnp.float32),
                pltpu.VMEM((2, _N), jnp.float32),
                pltpu.VMEM((_N, 2), jnp.float32),
                pltpu.VMEM((24, _B), jnp.float32),
            ],
        ),
        out_shape=jax.ShapeDtypeStruct((1, 1), jnp.float32),
        compiler_params=pltpu.CompilerParams(
            dimension_semantics=("arbitrary",)),
    )(jnp.asarray(_BS), jnp.asarray(_CS), sig_row, sig_all, y8, yt, py8, pyt)
    return out[0, 0]
